# baseline (device time: 12006 ns/iter reference)
import jax
import jax.numpy as jnp
from jax import lax
from jax.experimental import pallas as pl
from jax.experimental.pallas import tpu as pltpu

K = 4


def kernel(x):
    m, n = x.shape
    half = m // 2
    c = half // K

    def body(x_ref, out_ref, sbuf, r1buf, pbuf, r2buf,
             send1, recv1, send2, recv2):
        my_x = lax.axis_index("x")
        my_y = lax.axis_index("y")
        xn = (1 - my_x, my_y)
        yn = (my_x, 1 - my_y)

        barrier_sem = pltpu.get_barrier_semaphore()
        for nbr in (xn, yn):
            pl.semaphore_signal(
                barrier_sem, inc=1, device_id=nbr,
                device_id_type=pl.DeviceIdType.MESH,
            )
        pl.semaphore_wait(barrier_sem, 2)

        base = my_y * half
        other = (1 - my_y) * half

        sbuf[...] = x_ref[pl.ds(base, half), :].astype(jnp.bfloat16)
        rdma1 = []
        for k in range(K):
            r = pltpu.make_async_remote_copy(
                src_ref=sbuf.at[pl.ds(k * c, c), :],
                dst_ref=r1buf.at[pl.ds(k * c, c), :],
                send_sem=send1.at[k],
                recv_sem=recv1.at[k],
                device_id=xn,
                device_id_type=pl.DeviceIdType.MESH,
            )
            r.start()
            rdma1.append(r)

        rdma2 = []
        for k in range(K):
            ds = pl.ds(k * c, c)
            rdma1[k].wait_recv()
            pbuf[ds, :] = sbuf[ds, :] + r1buf[ds, :]
            r = pltpu.make_async_remote_copy(
                src_ref=pbuf.at[ds, :],
                dst_ref=r2buf.at[ds, :],
                send_sem=send2.at[k],
                recv_sem=recv2.at[k],
                device_id=yn,
                device_id_type=pl.DeviceIdType.MESH,
            )
            r.start()
            rdma2.append(r)
            out_ref[pl.ds(base + k * c, c), :] = pbuf[ds, :].astype(jnp.float32)

        for k in range(K):
            ds = pl.ds(k * c, c)
            rdma2[k].wait_recv()
            out_ref[pl.ds(other + k * c, c), :] = r2buf[ds, :].astype(jnp.float32)

        for k in range(K):
            rdma1[k].wait_send()
            rdma2[k].wait_send()

    return pl.pallas_call(
        body,
        out_shape=jax.ShapeDtypeStruct((m, n), jnp.float32),
        in_specs=[pl.BlockSpec(memory_space=pltpu.VMEM)],
        out_specs=pl.BlockSpec(memory_space=pltpu.VMEM),
        scratch_shapes=[
            pltpu.VMEM((half, n), jnp.bfloat16),
            pltpu.VMEM((half, n), jnp.bfloat16),
            pltpu.VMEM((half, n), jnp.bfloat16),
            pltpu.VMEM((half, n), jnp.bfloat16),
            pltpu.SemaphoreType.DMA((K,)),
            pltpu.SemaphoreType.DMA((K,)),
            pltpu.SemaphoreType.DMA((K,)),
            pltpu.SemaphoreType.DMA((K,)),
        ],
        compiler_params=pltpu.CompilerParams(collective_id=0),
    )(x)


# device time: 11491 ns/iter; 1.0448x vs baseline; 1.0448x over previous
import jax
import jax.numpy as jnp
from jax import lax
from jax.experimental import pallas as pl
from jax.experimental.pallas import tpu as pltpu

K = 8


def kernel(x):
    m, n = x.shape
    half = m // 2
    c = half // K

    def body(x_ref, out_ref, sbuf, r1buf, pbuf, r2buf,
             send1, recv1, send2, recv2):
        my_x = lax.axis_index("x")
        my_y = lax.axis_index("y")
        xn = (1 - my_x, my_y)
        yn = (my_x, 1 - my_y)

        barrier_sem = pltpu.get_barrier_semaphore()
        for nbr in (xn, yn):
            pl.semaphore_signal(
                barrier_sem, inc=1, device_id=nbr,
                device_id_type=pl.DeviceIdType.MESH,
            )
        pl.semaphore_wait(barrier_sem, 2)

        base = my_y * half
        other = (1 - my_y) * half

        rdma1 = []
        for k in range(K):
            ds = pl.ds(k * c, c)
            sbuf[ds, :] = x_ref[pl.ds(base + k * c, c), :].astype(jnp.bfloat16)
            r = pltpu.make_async_remote_copy(
                src_ref=sbuf.at[ds, :],
                dst_ref=r1buf.at[ds, :],
                send_sem=send1.at[k],
                recv_sem=recv1.at[k],
                device_id=xn,
                device_id_type=pl.DeviceIdType.MESH,
            )
            r.start()
            rdma1.append(r)

        rdma2 = []
        for k in range(K):
            ds = pl.ds(k * c, c)
            rdma1[k].wait_recv()
            pbuf[ds, :] = sbuf[ds, :] + r1buf[ds, :]
            r = pltpu.make_async_remote_copy(
                src_ref=pbuf.at[ds, :],
                dst_ref=r2buf.at[ds, :],
                send_sem=send2.at[k],
                recv_sem=recv2.at[k],
                device_id=yn,
                device_id_type=pl.DeviceIdType.MESH,
            )
            r.start()
            rdma2.append(r)
            out_ref[pl.ds(base + k * c, c), :] = pbuf[ds, :]

        for k in range(K):
            ds = pl.ds(k * c, c)
            rdma2[k].wait_recv()
            out_ref[pl.ds(other + k * c, c), :] = r2buf[ds, :]

        for k in range(K):
            rdma1[k].wait_send()
            rdma2[k].wait_send()

    return pl.pallas_call(
        body,
        out_shape=jax.ShapeDtypeStruct((m, n), jnp.bfloat16),
        in_specs=[pl.BlockSpec(memory_space=pltpu.VMEM)],
        out_specs=pl.BlockSpec(memory_space=pltpu.VMEM),
        scratch_shapes=[
            pltpu.VMEM((half, n), jnp.bfloat16),
            pltpu.VMEM((half, n), jnp.bfloat16),
            pltpu.VMEM((half, n), jnp.bfloat16),
            pltpu.VMEM((half, n), jnp.bfloat16),
            pltpu.SemaphoreType.DMA((K,)),
            pltpu.SemaphoreType.DMA((K,)),
            pltpu.SemaphoreType.DMA((K,)),
            pltpu.SemaphoreType.DMA((K,)),
        ],
        compiler_params=pltpu.CompilerParams(collective_id=0),
    )(x)
